# baseline (device time: 16632 ns/iter reference)
import jax
import jax.numpy as jnp
from jax import lax
from jax.experimental import pallas as pl
from jax.experimental.pallas import tpu as pltpu

N_DEV = 4
B, Sq, Skv, Hq, Dh = 2, 256, 256, 16, 64
H_LOC = Hq // N_DEV
M, N = B * Sq, 512
C = M // N_DEV

_bf = jnp.bfloat16
_f32 = jnp.float32


def _fused(x2, Wq, Kl, Vl, Wo):
    def body(x_ref, wq_ref, k_ref, v_ref, wo_ref, out_ref,
             chunks, rs_buf, red_ref, wq_bf, wo_bf,
             rs_send, rs_recv, ag_send, ag_recv):
        my = lax.axis_index("i")

        barrier_sem = pltpu.get_barrier_semaphore()
        for j in range(1, N_DEV):
            pl.semaphore_signal(
                barrier_sem, inc=1,
                device_id=(lax.rem(my + j, N_DEV),),
                device_id_type=pl.DeviceIdType.MESH,
            )

        wq_bf[:, :] = (wq_ref[:, :] * 0.125).astype(_bf)
        wo_bf[:, :] = wo_ref[:, :].astype(_bf)

        def compute_chunk(j):
            b = lax.div(j, 2)
            qo = lax.rem(j, 2) * C
            xc = x_ref[pl.ds(j * C, C), :].astype(_bf)
            qc = jnp.dot(xc, wq_bf[:, :],
                         preferred_element_type=_f32).astype(_bf)
            qb = (qo + lax.broadcasted_iota(jnp.int32, (C, Skv), 0)) // 64
            kb = lax.broadcasted_iota(jnp.int32, (C, Skv), 1) // 64
            maskf = ((qb == kb) | (kb == 0) | (lax.rem(qb + kb, 3) == 0)
                     ).astype(_f32)
            q4 = jnp.stack(
                [qc[:, h * Dh:(h + 1) * Dh] for h in range(H_LOC)]
            )
            k4 = k_ref[pl.ds(b * H_LOC, H_LOC), :, :]
            v4 = v_ref[pl.ds(b * H_LOC, H_LOC), :, :]
            s4 = lax.dot_general(
                q4, k4, (((2,), (2,)), ((0,), (0,))),
                preferred_element_type=_f32,
            )
            w4 = jnp.exp(s4) * maskf[None]
            w4 = w4 * (1.0 / jnp.sum(w4, axis=-1, keepdims=True))
            ctx4 = lax.dot_general(
                w4.astype(_bf), v4, (((2,), (1,)), ((0,), (0,))),
                preferred_element_type=_f32,
            )
            pc = jnp.zeros((C, N), _f32)
            for h in range(H_LOC):
                pc = pc + jnp.dot(
                    ctx4[h].astype(_bf), wo_bf[h * Dh:(h + 1) * Dh, :],
                    preferred_element_type=_f32)
            return pc

        send_descs = []
        for t in range(N_DEV - 1):
            dst = lax.rem(my + 1 + t, N_DEV)
            pc = compute_chunk(dst)
            chunks[dst, :, :] = pc.astype(_bf)
            if t == 0:
                pl.semaphore_wait(barrier_sem, N_DEV - 1)
            rdma = pltpu.make_async_remote_copy(
                src_ref=chunks.at[dst],
                dst_ref=rs_buf.at[my],
                send_sem=rs_send.at[t],
                recv_sem=rs_recv.at[my],
                device_id=(dst,),
                device_id_type=pl.DeviceIdType.MESH,
            )
            rdma.start()
            send_descs.append(rdma)

        acc = compute_chunk(my)

        for s in range(N_DEV):
            recv = pltpu.make_async_remote_copy(
                src_ref=chunks.at[s], dst_ref=rs_buf.at[s],
                send_sem=rs_send.at[0], recv_sem=rs_recv.at[s],
                device_id=(s,), device_id_type=pl.DeviceIdType.MESH,
            )
            pl.when(my != s)(recv.wait_recv)
            acc = acc + jnp.where(my == s, 0.0, rs_buf[s, :, :].astype(_f32))
        red_ref[:, :] = acc.astype(_bf)
        out_ref[pl.ds(my * C, C), :] = red_ref[:, :]

        for t in (1, 0, 2):
            dst = lax.rem(my + 1 + t, N_DEV)
            rdma = pltpu.make_async_remote_copy(
                src_ref=red_ref,
                dst_ref=out_ref.at[pl.ds(my * C, C)],
                send_sem=ag_send.at[t],
                recv_sem=ag_recv.at[my],
                device_id=(dst,),
                device_id_type=pl.DeviceIdType.MESH,
            )
            rdma.start()
            send_descs.append(rdma)

        for s in range(N_DEV):
            recv = pltpu.make_async_remote_copy(
                src_ref=red_ref, dst_ref=out_ref.at[pl.ds(s * C, C)],
                send_sem=ag_send.at[0], recv_sem=ag_recv.at[s],
                device_id=(s,), device_id_type=pl.DeviceIdType.MESH,
            )
            pl.when(my != s)(recv.wait_recv)

        for rdma in send_descs:
            rdma.wait_send()

    return pl.pallas_call(
        body,
        out_shape=jax.ShapeDtypeStruct((M, N), _bf),
        in_specs=[pl.BlockSpec(memory_space=pltpu.VMEM)] * 5,
        out_specs=pl.BlockSpec(memory_space=pltpu.VMEM),
        scratch_shapes=[
            pltpu.VMEM((N_DEV, C, N), _bf),
            pltpu.VMEM((N_DEV, C, N), _bf),
            pltpu.VMEM((C, N), _bf),
            pltpu.VMEM((512, 256), _bf),
            pltpu.VMEM((256, 512), _bf),
            pltpu.SemaphoreType.DMA((N_DEV - 1,)),
            pltpu.SemaphoreType.DMA((N_DEV,)),
            pltpu.SemaphoreType.DMA((N_DEV - 1,)),
            pltpu.SemaphoreType.DMA((N_DEV,)),
        ],
        compiler_params=pltpu.CompilerParams(collective_id=0),
    )(x2, Wq, Kl, Vl, Wo)


def kernel(x, Wq, K_ext, V_ext, Wo):
    my = lax.axis_index("i")
    x2 = x.reshape(M, -1)
    Kl = lax.dynamic_slice_in_dim(K_ext, my * H_LOC, H_LOC, axis=2)
    Vl = lax.dynamic_slice_in_dim(V_ext, my * H_LOC, H_LOC, axis=2)
    Kl = Kl.astype(_bf).transpose(0, 2, 1, 3).reshape(B * H_LOC, Skv, Dh)
    Vl = Vl.astype(_bf).transpose(0, 2, 1, 3).reshape(B * H_LOC, Skv, Dh)
    out = _fused(x2, Wq, Kl, Vl, Wo)
    return out.reshape(B, Sq, -1)


# device time: 15788 ns/iter; 1.0535x vs baseline; 1.0535x over previous
import jax
import jax.numpy as jnp
from jax import lax
from jax.experimental import pallas as pl
from jax.experimental.pallas import tpu as pltpu

N_DEV = 4
B, Sq, Skv, Hq, Dh = 2, 256, 256, 16, 64
H_LOC = Hq // N_DEV
M, N = B * Sq, 512
C = M // N_DEV

_bf = jnp.bfloat16
_f32 = jnp.float32


def _fused(x2, Wq, Kl, Vl, Wo):
    def body(x_ref, wq_ref, k_ref, v_ref, wo_ref, out_ref,
             chunks, rs_buf, red_ref, wq_bf, wo_bf,
             rs_send, rs_recv, ag_send, ag_recv):
        my = lax.axis_index("i")

        barrier_sem = pltpu.get_barrier_semaphore()
        for j in range(1, N_DEV):
            pl.semaphore_signal(
                barrier_sem, inc=1,
                device_id=(lax.rem(my + j, N_DEV),),
                device_id_type=pl.DeviceIdType.MESH,
            )

        wq_bf[:, :] = (wq_ref[:, :] * 0.125).astype(_bf)
        wo_bf[:, :] = wo_ref[:, :].astype(_bf)

        def compute_chunk(j):
            b = lax.div(j, 2)
            qo = lax.rem(j, 2) * C
            xc = x_ref[pl.ds(j * C, C), :].astype(_bf)
            qc = jnp.dot(xc, wq_bf[:, :],
                         preferred_element_type=_f32).astype(_bf)
            qb = (qo + lax.broadcasted_iota(jnp.int32, (C, Skv), 0)) // 64
            kb = lax.broadcasted_iota(jnp.int32, (C, Skv), 1) // 64
            maskf = ((qb == kb) | (kb == 0) | (lax.rem(qb + kb, 3) == 0)
                     ).astype(_f32)
            q4 = jnp.stack(
                [qc[:, h * Dh:(h + 1) * Dh] for h in range(H_LOC)]
            )
            k4 = k_ref[pl.ds(b * H_LOC, H_LOC), :, :]
            v4 = v_ref[pl.ds(b * H_LOC, H_LOC), :, :]
            s4 = lax.dot_general(
                q4, k4, (((2,), (2,)), ((0,), (0,))),
                preferred_element_type=_f32,
            )
            w4 = jnp.exp(s4) * maskf[None]
            w4 = w4 * (1.0 / jnp.sum(w4, axis=-1, keepdims=True))
            ctx4 = lax.dot_general(
                w4.astype(_bf), v4, (((2,), (1,)), ((0,), (0,))),
                preferred_element_type=_f32,
            )
            pc = jnp.zeros((C, N), _f32)
            for h in range(H_LOC):
                pc = pc + jnp.dot(
                    ctx4[h].astype(_bf), wo_bf[h * Dh:(h + 1) * Dh, :],
                    preferred_element_type=_f32)
            return pc

        send_descs = []
        for t in range(N_DEV - 1):
            dst = lax.rem(my + 1 + t, N_DEV)
            pc = compute_chunk(dst)
            chunks[dst, :, :] = pc.astype(_bf)
            if t == 0:
                pl.semaphore_wait(barrier_sem, N_DEV - 1)
            rdma = pltpu.make_async_remote_copy(
                src_ref=chunks.at[dst],
                dst_ref=rs_buf.at[my],
                send_sem=rs_send.at[t],
                recv_sem=rs_recv.at[my],
                device_id=(dst,),
                device_id_type=pl.DeviceIdType.MESH,
            )
            rdma.start()
            send_descs.append(rdma)

        acc = compute_chunk(my)

        for s in range(N_DEV):
            recv = pltpu.make_async_remote_copy(
                src_ref=chunks.at[s], dst_ref=rs_buf.at[s],
                send_sem=rs_send.at[0], recv_sem=rs_recv.at[s],
                device_id=(s,), device_id_type=pl.DeviceIdType.MESH,
            )
            pl.when(my != s)(recv.wait_recv)
            acc = acc + jnp.where(my == s, 0.0, rs_buf[s, :, :].astype(_f32))
        red_ref[:, :] = acc.astype(_bf)
        out_ref[pl.ds(my * C, C), :] = red_ref[:, :]

        for t in range(N_DEV - 1):
            dst = lax.rem(my + 1 + t, N_DEV)
            rdma = pltpu.make_async_remote_copy(
                src_ref=red_ref,
                dst_ref=out_ref.at[pl.ds(my * C, C)],
                send_sem=ag_send.at[t],
                recv_sem=ag_recv.at[my],
                device_id=(dst,),
                device_id_type=pl.DeviceIdType.MESH,
            )
            rdma.start()
            send_descs.append(rdma)

        for s in range(N_DEV):
            recv = pltpu.make_async_remote_copy(
                src_ref=red_ref, dst_ref=out_ref.at[pl.ds(s * C, C)],
                send_sem=ag_send.at[0], recv_sem=ag_recv.at[s],
                device_id=(s,), device_id_type=pl.DeviceIdType.MESH,
            )
            pl.when(my != s)(recv.wait_recv)

        for rdma in send_descs:
            rdma.wait_send()

    return pl.pallas_call(
        body,
        out_shape=jax.ShapeDtypeStruct((M, N), _bf),
        in_specs=[pl.BlockSpec(memory_space=pltpu.VMEM)] * 5,
        out_specs=pl.BlockSpec(memory_space=pltpu.VMEM),
        scratch_shapes=[
            pltpu.VMEM((N_DEV, C, N), _bf),
            pltpu.VMEM((N_DEV, C, N), _bf),
            pltpu.VMEM((C, N), _bf),
            pltpu.VMEM((512, 256), _bf),
            pltpu.VMEM((256, 512), _bf),
            pltpu.SemaphoreType.DMA((N_DEV - 1,)),
            pltpu.SemaphoreType.DMA((N_DEV,)),
            pltpu.SemaphoreType.DMA((N_DEV - 1,)),
            pltpu.SemaphoreType.DMA((N_DEV,)),
        ],
        compiler_params=pltpu.CompilerParams(collective_id=0),
    )(x2, Wq, Kl, Vl, Wo)


def kernel(x, Wq, K_ext, V_ext, Wo):
    my = lax.axis_index("i")
    x2 = x.reshape(M, -1)
    Kl = lax.dynamic_slice_in_dim(K_ext, my * H_LOC, H_LOC, axis=2)
    Vl = lax.dynamic_slice_in_dim(V_ext, my * H_LOC, H_LOC, axis=2)
    Kl = Kl.astype(_bf).transpose(0, 2, 1, 3).reshape(B * H_LOC, Skv, Dh)
    Vl = Vl.astype(_bf).transpose(0, 2, 1, 3).reshape(B * H_LOC, Skv, Dh)
    out = _fused(x2, Wq, Kl, Vl, Wo)
    return out.reshape(B, Sq, -1)
